# baseline (speedup 1.0000x reference)
import functools

import jax
import jax.numpy as jnp
from jax import lax
from jax.experimental import pallas as pl
from jax.experimental.pallas import tpu as pltpu
from jax.experimental.pallas import tpu_sc as plsc

_R = 128
_C = 32768
_G = 256
_GPR = _C // _G
_N = _R * _GPR

_CR = 8
_CC = 4096
_NSB = 8
_NCH = 4
_GPW = 512
_H = 128

_mesh = plsc.VectorSubcoreMesh(core_axis_name="c", subcore_axis_name="s")


@functools.partial(
    pl.kernel,
    mesh=_mesh,
    out_type=jax.ShapeDtypeStruct((_N,), jnp.int32),
    scratch_types=[
        pltpu.VMEM((_CR, _CC), jnp.float32),
        pltpu.VMEM((_CR, _CC), jnp.float32),
        pltpu.VMEM((_GPW,), jnp.int32),
        pltpu.VMEM((_GPW,), jnp.int32),
        pltpu.SemaphoreType.DMA,
        pltpu.SemaphoreType.DMA,
        pltpu.SemaphoreType.DMA,
    ],
    compiler_params=pltpu.CompilerParams(needs_layout_passes=False),
)
def _sc_idx(x_hbm, idx_hbm, in0, in1, idxbuf, posbuf, si0, si1, so):
    ins, isems = [in0, in1], [si0, si1]
    wid = lax.axis_index("s") * 2 + lax.axis_index("c")
    r0 = (wid % 16) * _CR
    gc0 = (wid // 16) * (_NCH * _CC // _G)
    lane = lax.iota(jnp.int32, 16)

    row_idx = jnp.bitwise_and(lane, 7)
    lgc = jnp.right_shift(lane, 3)
    colb = [(lgc + 2 * b) * _G for b in range(_NSB)]
    neginf = jnp.full((16,), -jnp.inf, jnp.float32)
    izero = jnp.zeros((16,), jnp.int32)

    def pos_body(i, _):
        p = i * 16 + lane
        gpos = (r0 + jnp.right_shift(p, 6)) * _GPR + gc0 + jnp.bitwise_and(p, 63)
        posbuf[pl.ds(i * 16, 16)] = gpos
        return _

    lax.fori_loop(0, _GPW // 16, pos_body, None)

    def make_scan(buf):
        def scan_elems(e, carry):
            curs, idxs = carry
            nc, ni = [], []
            for b in range(_NSB):
                v = plsc.load_gather(buf, [row_idx, colb[b] + e])
                upd = v > curs[b]
                nc.append(jnp.maximum(curs[b], v))
                ni.append(jnp.where(upd, e, idxs[b]))
            return tuple(nc), tuple(ni)
        return scan_elems

    def start_in(c, b):
        cc = (gc0 + c * (_CC // _G)) * _G
        return pltpu.async_copy(
            x_hbm.at[pl.ds(r0, _CR), pl.ds(cc, _CC)], ins[b], isems[b])

    init = (tuple(neginf for _ in range(_NSB)),
            tuple(izero for _ in range(_NSB)))
    in_h = {0: start_in(0, 0)}
    for c in range(_NCH):
        b = c & 1
        if c + 1 < _NCH:
            in_h[c + 1] = start_in(c + 1, 1 - b)
        in_h[c].wait()
        curs, idxs = plsc.parallel_loop(
            0, _G, carry=init, unroll=8)(make_scan(ins[b]))
        for sb in range(_NSB):
            gidx = idxs[sb]
            p = row_idx * (_GPW // _CR) + (c * (_CC // _G) + 2 * sb + lgc)
            plsc.store_scatter(idxbuf, [p], gidx)
    pltpu.async_copy(idxbuf, idx_hbm.at[posbuf], so).wait()


_BR = 8


def _tc_body(idx_ref, o_ref):
    iota = lax.broadcasted_iota(jnp.int32, (_BR, _G), 1).astype(jnp.float32)
    idx_f = idx_ref[...].astype(jnp.float32)
    for k in range(_GPR):
        col = idx_f[:, k:k + 1]
        o_ref[:, k * _G:(k + 1) * _G] = jnp.where(iota == col, 1.0, 0.0)


def kernel(x):
    idx = _sc_idx(x)
    idx2d = idx.reshape(_R, _GPR)
    return pl.pallas_call(
        _tc_body,
        grid=(_R // _BR,),
        in_specs=[pl.BlockSpec((_BR, _GPR), lambda i: (i, 0))],
        out_specs=pl.BlockSpec((_BR, _C), lambda i: (i, 0)),
        out_shape=jax.ShapeDtypeStruct((_R, _C), jnp.float32),
    )(idx2d)


# speedup vs baseline: 1.5375x; 1.5375x over previous
import functools

import jax
import jax.numpy as jnp
from jax import lax
from jax.experimental import pallas as pl
from jax.experimental.pallas import tpu as pltpu
from jax.experimental.pallas import tpu_sc as plsc

_R = 128
_C = 32768
_G = 256
_GPR = _C // _G
_N = _R * _GPR

_CR = 8
_CC = 4096
_GPC = _CC // _G
_NCH = 4
_GPW = 512

_mesh = plsc.VectorSubcoreMesh(core_axis_name="c", subcore_axis_name="s")


_GDN = lax.GatherDimensionNumbers(
    offset_dims=(), collapsed_slice_dims=(0,), start_index_map=(0,))


def _take(v, perm):
    return lax.gather(v, perm[:, None], _GDN, (1,),
                      mode=lax.GatherScatterMode.PROMISE_IN_BOUNDS)


@functools.partial(
    pl.kernel,
    mesh=_mesh,
    out_type=jax.ShapeDtypeStruct((_N,), jnp.int32),
    scratch_types=[
        pltpu.VMEM((_CR, _CC), jnp.float32),
        pltpu.VMEM((_CR, _CC), jnp.float32),
        pltpu.VMEM((_GPW,), jnp.int32),
        pltpu.VMEM((_GPW,), jnp.int32),
        pltpu.SemaphoreType.DMA,
        pltpu.SemaphoreType.DMA,
        pltpu.SemaphoreType.DMA,
    ],
    compiler_params=pltpu.CompilerParams(needs_layout_passes=False),
)
def _sc_idx(x_hbm, idx_hbm, in0, in1, idxbuf, posbuf, si0, si1, so):
    ins, isems = [in0, in1], [si0, si1]
    wid = lax.axis_index("s") * 2 + lax.axis_index("c")
    r0 = (wid % 16) * _CR
    gc0 = (wid // 16) * (_NCH * _GPC)
    lane = lax.iota(jnp.int32, 16)
    perms = [jnp.bitwise_xor(lane, s) for s in (1, 2, 4, 8)]

    def pos_body(i, _):
        p = i * 16 + lane
        gpos = (r0 + jnp.right_shift(p, 6)) * _GPR + gc0 + jnp.bitwise_and(p, 63)
        posbuf[pl.ds(i * 16, 16)] = gpos
        return _

    lax.fori_loop(0, _GPW // 16, pos_body, None)

    def make_group(buf, r):
        def group_body(gc, results):
            base = gc * _G
            cur = buf[r, pl.ds(base, 16)]
            curj = jnp.zeros((16,), jnp.int32)
            for j in range(1, 16):
                v = buf[r, pl.ds(base + j * 16, 16)]
                upd = v > cur
                cur = jnp.maximum(cur, v)
                curj = jnp.where(upd, j, curj)
            idx = curj * 16 + lane
            for perm in perms:
                pc = _take(cur, perm)
                pi = _take(idx, perm)
                take = (pc > cur) | ((pc == cur) & (pi < idx))
                cur = jnp.where(take, pc, cur)
                idx = jnp.where(take, pi, idx)
            return jnp.where(lane == gc, idx, results)
        return group_body

    def start_in(c, b):
        cc = (gc0 + c * _GPC) * _G
        return pltpu.async_copy(
            x_hbm.at[pl.ds(r0, _CR), pl.ds(cc, _CC)], ins[b], isems[b])

    in_h = {0: start_in(0, 0)}
    for c in range(_NCH):
        b = c & 1
        if c + 1 < _NCH:
            in_h[c + 1] = start_in(c + 1, 1 - b)
        in_h[c].wait()
        for r in range(_CR):
            res = plsc.parallel_loop(
                0, _GPC, carry=jnp.zeros((16,), jnp.int32), unroll=2)(
                    make_group(ins[b], r))
            idxbuf[pl.ds(r * (_GPW // _CR) + c * _GPC, 16)] = res
    pltpu.async_copy(idxbuf, idx_hbm.at[posbuf], so).wait()


_BR = 8


def _tc_body(idx_ref, o_ref):
    iota = lax.broadcasted_iota(jnp.int32, (_BR, _G), 1).astype(jnp.float32)
    idx_f = idx_ref[...].astype(jnp.float32)
    for k in range(_GPR):
        col = idx_f[:, k:k + 1]
        o_ref[:, k * _G:(k + 1) * _G] = jnp.where(iota == col, 1.0, 0.0)


def kernel(x):
    idx = _sc_idx(x)
    idx2d = idx.reshape(_R, _GPR)
    return pl.pallas_call(
        _tc_body,
        grid=(_R // _BR,),
        in_specs=[pl.BlockSpec((_BR, _GPR), lambda i: (i, 0))],
        out_specs=pl.BlockSpec((_BR, _C), lambda i: (i, 0)),
        out_shape=jax.ShapeDtypeStruct((_R, _C), jnp.float32),
    )(idx2d)
